# baseline (speedup 1.0000x reference)
import functools

import jax
import jax.numpy as jnp
from jax import lax
from jax.experimental import pallas as pl
from jax.experimental.pallas import tpu as pltpu
from jax.experimental.pallas import tpu_sc as plsc

_R = 100000
_C = 128
_B = 16384
_N = _B * _C
_NW = 32
_PER_W = _N // _NW
_CH = 16384
_NCHUNK = _PER_W // _CH
_L = 16


def _sc_gather(idx_flat, data_flat):
    mesh = plsc.VectorSubcoreMesh(core_axis_name="c", subcore_axis_name="s")

    scratch = (
        [pltpu.VMEM((_CH,), jnp.int32) for _ in range(2)]
        + [pltpu.VMEM((_CH,), jnp.float32) for _ in range(_NCHUNK)]
        + [pltpu.SemaphoreType.DMA for _ in range(2 * _NCHUNK)]
    )

    @functools.partial(
        pl.kernel,
        mesh=mesh,
        out_type=jax.ShapeDtypeStruct((_N,), jnp.float32),
        scratch_types=scratch,
    )
    def k(idx_hbm, data_hbm, out_hbm, *scr):
        idx_bufs = scr[:2]
        val_bufs = scr[2:2 + _NCHUNK]
        gsems = scr[2 + _NCHUNK:2 + 2 * _NCHUNK]
        osems = scr[2 + 2 * _NCHUNK:]
        wid = lax.axis_index("s") * 2 + lax.axis_index("c")
        base = wid * _PER_W
        lanes = lax.iota(jnp.int32, _L)
        col_vecs = [col0 + lanes for col0 in range(0, _C, _L)]

        def stage(g):
            b = g & 1
            buf = idx_bufs[b]
            cbase = base + g * _CH
            pltpu.sync_copy(idx_hbm.at[pl.ds(cbase, _CH)], buf)

            def vec_body(o, carry):
                boff = pl.multiple_of(o * _C, _C)
                for t in range(_C // _L):
                    off = boff + t * _L
                    v = buf[pl.ds(off, _L)]
                    buf[pl.ds(off, _L)] = (v << 7) + col_vecs[t]
                return carry

            lax.fori_loop(0, _CH // _C, vec_body, 0, unroll=2)
            return pltpu.async_copy(data_hbm.at[buf], val_bufs[g], gsems[g])

        gdescs = [None] * _NCHUNK
        odescs = [None] * _NCHUNK

        def drain(g):
            gdescs[g].wait()
            odescs[g] = pltpu.async_copy(
                val_bufs[g], out_hbm.at[pl.ds(base + g * _CH, _CH)], osems[g])

        for g in range(_NCHUNK):
            if g >= 2:
                drain(g - 2)
            gdescs[g] = stage(g)
        for g in range(_NCHUNK - 2, _NCHUNK):
            drain(g)
        for g in range(_NCHUNK):
            odescs[g].wait()

    return k(idx_flat, data_flat)


def kernel(data, indices, axis):
    del axis
    out_flat = _sc_gather(indices.reshape(-1), data.reshape(-1))
    return out_flat.reshape(_B, _C)


# speedup vs baseline: 1.0052x; 1.0052x over previous
import functools

import jax
import jax.numpy as jnp
from jax import lax
from jax.experimental import pallas as pl
from jax.experimental.pallas import tpu as pltpu
from jax.experimental.pallas import tpu_sc as plsc

_R = 100000
_C = 128
_B = 16384
_N = _B * _C
_NW = 32
_PER_W = _N // _NW
_CH = 16384
_NCHUNK = _PER_W // _CH
_L = 16


def _sc_gather(idx_flat, data_flat):
    mesh = plsc.VectorSubcoreMesh(core_axis_name="c", subcore_axis_name="s")

    scratch = (
        [pltpu.VMEM((_PER_W,), jnp.int32)]
        + [pltpu.VMEM((_CH,), jnp.float32) for _ in range(_NCHUNK)]
        + [pltpu.SemaphoreType.DMA for _ in range(1 + 2 * _NCHUNK)]
    )

    @functools.partial(
        pl.kernel,
        mesh=mesh,
        out_type=jax.ShapeDtypeStruct((_N,), jnp.float32),
        scratch_types=scratch,
    )
    def k(idx_hbm, data_hbm, out_hbm, *scr):
        idx_v = scr[0]
        val_bufs = scr[1:1 + _NCHUNK]
        isem = scr[1 + _NCHUNK]
        gsems = scr[2 + _NCHUNK:2 + 2 * _NCHUNK]
        osems = scr[2 + 2 * _NCHUNK:]
        wid = lax.axis_index("s") * 2 + lax.axis_index("c")
        base = wid * _PER_W
        lanes = lax.iota(jnp.int32, _L)
        col_vecs = [col0 + lanes for col0 in range(0, _C, _L)]

        pltpu.async_copy(idx_hbm.at[pl.ds(base, _PER_W)], idx_v, isem).wait()

        gdescs = [None] * _NCHUNK
        for g in range(_NCHUNK):
            def vec_body(o, carry, _g=g):
                boff = pl.multiple_of(_g * _CH + o * _C, _C)
                for t in range(_C // _L):
                    off = boff + t * _L
                    v = idx_v[pl.ds(off, _L)]
                    idx_v[pl.ds(off, _L)] = (v << 7) + col_vecs[t]
                return carry

            lax.fori_loop(0, _CH // _C, vec_body, 0, unroll=2)
            gdescs[g] = pltpu.async_copy(
                data_hbm.at[idx_v.at[pl.ds(g * _CH, _CH)]],
                val_bufs[g], gsems[g])

        odescs = [None] * _NCHUNK
        for g in range(_NCHUNK):
            gdescs[g].wait()
            odescs[g] = pltpu.async_copy(
                val_bufs[g], out_hbm.at[pl.ds(base + g * _CH, _CH)], osems[g])
        for g in range(_NCHUNK):
            odescs[g].wait()

    return k(idx_flat, data_flat)


def kernel(data, indices, axis):
    del axis
    out_flat = _sc_gather(indices.reshape(-1), data.reshape(-1))
    return out_flat.reshape(_B, _C)


# speedup vs baseline: 1.0065x; 1.0012x over previous
import functools

import jax
import jax.numpy as jnp
from jax import lax
from jax.experimental import pallas as pl
from jax.experimental.pallas import tpu as pltpu
from jax.experimental.pallas import tpu_sc as plsc

_R = 100000
_C = 128
_B = 16384
_N = _B * _C
_NW = 32
_PER_W = _N // _NW
_CH = 8192
_NCHUNK = _PER_W // _CH
_L = 16


def _sc_gather(idx_flat, data_flat):
    mesh = plsc.VectorSubcoreMesh(core_axis_name="c", subcore_axis_name="s")

    scratch = (
        [pltpu.VMEM((_PER_W,), jnp.int32)]
        + [pltpu.VMEM((_CH,), jnp.float32) for _ in range(_NCHUNK)]
        + [pltpu.SemaphoreType.DMA for _ in range(1 + 2 * _NCHUNK)]
    )

    @functools.partial(
        pl.kernel,
        mesh=mesh,
        out_type=jax.ShapeDtypeStruct((_N,), jnp.float32),
        scratch_types=scratch,
    )
    def k(idx_hbm, data_hbm, out_hbm, *scr):
        idx_v = scr[0]
        val_bufs = scr[1:1 + _NCHUNK]
        isem = scr[1 + _NCHUNK]
        gsems = scr[2 + _NCHUNK:2 + 2 * _NCHUNK]
        osems = scr[2 + 2 * _NCHUNK:]
        wid = lax.axis_index("s") * 2 + lax.axis_index("c")
        base = wid * _PER_W
        lanes = lax.iota(jnp.int32, _L)
        col_vecs = [col0 + lanes for col0 in range(0, _C, _L)]

        pltpu.async_copy(idx_hbm.at[pl.ds(base, _PER_W)], idx_v, isem).wait()

        gdescs = [None] * _NCHUNK
        for g in range(_NCHUNK):
            def vec_body(o, carry, _g=g):
                boff = pl.multiple_of(_g * _CH + o * _C, _C)
                for t in range(_C // _L):
                    off = boff + t * _L
                    v = idx_v[pl.ds(off, _L)]
                    idx_v[pl.ds(off, _L)] = (v << 7) + col_vecs[t]
                return carry

            lax.fori_loop(0, _CH // _C, vec_body, 0, unroll=2)
            gdescs[g] = pltpu.async_copy(
                data_hbm.at[idx_v.at[pl.ds(g * _CH, _CH)]],
                val_bufs[g], gsems[g])

        odescs = [None] * _NCHUNK
        for g in range(_NCHUNK):
            gdescs[g].wait()
            odescs[g] = pltpu.async_copy(
                val_bufs[g], out_hbm.at[pl.ds(base + g * _CH, _CH)], osems[g])
        for g in range(_NCHUNK):
            odescs[g].wait()

    return k(idx_flat, data_flat)


def kernel(data, indices, axis):
    del axis
    out_flat = _sc_gather(indices.reshape(-1), data.reshape(-1))
    return out_flat.reshape(_B, _C)


# speedup vs baseline: 1.0194x; 1.0128x over previous
import functools

import jax
import jax.numpy as jnp
from jax import lax
from jax.experimental import pallas as pl
from jax.experimental.pallas import tpu as pltpu
from jax.experimental.pallas import tpu_sc as plsc

_R = 100000
_C = 128
_B = 16384
_N = _B * _C
_NW = 32
_PER_W = _N // _NW
_CH = 8192
_NCHUNK = _PER_W // _CH
_L = 16


def _sc_gather(idx_flat, data_flat):
    mesh = plsc.VectorSubcoreMesh(core_axis_name="c", subcore_axis_name="s")

    _NVAL = 4

    scratch = (
        [pltpu.VMEM((_CH,), jnp.int32) for _ in range(_NCHUNK)]
        + [pltpu.VMEM((_CH,), jnp.float32) for _ in range(_NVAL)]
        + [pltpu.SemaphoreType.DMA for _ in range(3 * _NCHUNK)]
    )

    @functools.partial(
        pl.kernel,
        mesh=mesh,
        out_type=jax.ShapeDtypeStruct((_N,), jnp.float32),
        scratch_types=scratch,
    )
    def k(idx_hbm, data_hbm, out_hbm, *scr):
        idx_bufs = scr[:_NCHUNK]
        val_bufs = scr[_NCHUNK:_NCHUNK + _NVAL]
        isems = scr[_NCHUNK + _NVAL:2 * _NCHUNK + _NVAL]
        gsems = scr[2 * _NCHUNK + _NVAL:3 * _NCHUNK + _NVAL]
        osems = scr[3 * _NCHUNK + _NVAL:]
        wid = lax.axis_index("s") * 2 + lax.axis_index("c")
        base = wid * _PER_W
        lanes = lax.iota(jnp.int32, _L)
        col_vecs = [col0 + lanes for col0 in range(0, _C, _L)]

        def load_idx(g):
            return pltpu.async_copy(
                idx_hbm.at[pl.ds(base + g * _CH, _CH)], idx_bufs[g], isems[g])

        ildescs = [None] * _NCHUNK
        gdescs = [None] * _NCHUNK
        odescs = [None] * _NCHUNK
        ildescs[0] = load_idx(0)
        ildescs[1] = load_idx(1)

        for g in range(_NCHUNK):
            ildescs[g].wait()
            buf = idx_bufs[g]

            def vec_body(o, carry, _buf=buf):
                boff = pl.multiple_of(o * _C, _C)
                for t in range(_C // _L):
                    off = boff + t * _L
                    v = _buf[pl.ds(off, _L)]
                    _buf[pl.ds(off, _L)] = (v << 7) + col_vecs[t]
                return carry

            lax.fori_loop(0, _CH // _C, vec_body, 0, unroll=2)
            if g >= _NVAL:
                odescs[g - _NVAL].wait()
            gdescs[g] = pltpu.async_copy(
                data_hbm.at[buf], val_bufs[g % _NVAL], gsems[g])
            if g + 2 < _NCHUNK:
                ildescs[g + 2] = load_idx(g + 2)
            if g >= 1:
                gdescs[g - 1].wait()
                odescs[g - 1] = pltpu.async_copy(
                    val_bufs[(g - 1) % _NVAL],
                    out_hbm.at[pl.ds(base + (g - 1) * _CH, _CH)],
                    osems[g - 1])

        g = _NCHUNK - 1
        gdescs[g].wait()
        odescs[g] = pltpu.async_copy(
            val_bufs[g % _NVAL], out_hbm.at[pl.ds(base + g * _CH, _CH)],
            osems[g])
        for g in range(_NCHUNK - _NVAL, _NCHUNK):
            odescs[g].wait()

    return k(idx_flat, data_flat)


def kernel(data, indices, axis):
    del axis
    out_flat = _sc_gather(indices.reshape(-1), data.reshape(-1))
    return out_flat.reshape(_B, _C)
